# Initial kernel scaffold; baseline (speedup 1.0000x reference)
import jax
import jax.numpy as jnp
from jax.experimental import pallas as pl


def kernel(objects, relations, senders, receivers, rW1, rb1, rW2, rb2, oW1, ob1, oW2, ob2, eW1, eb1, eW2, eb2):
    raise NotImplementedError("write your pallas kernel here")


# speedup vs baseline: 2.6822x; 2.6822x over previous
import functools

import jax
import jax.numpy as jnp
from jax import lax
from jax.experimental import pallas as pl
from jax.experimental.pallas import tpu as pltpu
from jax.experimental.pallas import tpu_sc as plsc

_N = 10000
_E = 320000
_OD = 128
_RD = 16

_NC = 2
_NS = 16
_NW = _NC * _NS
_EW = _E // _NW
_C = 80
_NCHUNK = _EW // _C
_ROWS_PER_TILE = 624
_TAIL_ROWS = _N - _NS * _ROWS_PER_TILE


def _pre_body(obj_ref, rw1a_ref, rw1b_ref, rb1_ref, ew1a_ref, eb1_ref,
              p_ref, q_ref, u_ref):
    obj = obj_ref[...]
    p_ref[...] = jnp.dot(obj, rw1a_ref[...], preferred_element_type=jnp.float32)
    q_ref[...] = (jnp.dot(obj, rw1b_ref[...], preferred_element_type=jnp.float32)
                  + rb1_ref[...])
    u_ref[...] = (jnp.dot(obj, ew1a_ref[...], preferred_element_type=jnp.float32)
                  + eb1_ref[...])


def _gather_body(p_hbm, q_hbm, snd_hbm, rcv_hbm, pg_hbm, qg_hbm,
                 sidx, ridx, prow, qrow, sem_p, sem_q):
    c = lax.axis_index("c")
    s = lax.axis_index("s")
    wid = s * _NC + c
    base = wid * _EW

    def body(j, carry):
        off = base + j * _C
        pltpu.sync_copy(snd_hbm.at[pl.ds(off, _C)], sidx)
        pltpu.sync_copy(rcv_hbm.at[pl.ds(off, _C)], ridx)
        cp_p = pltpu.async_copy(p_hbm.at[sidx], prow, sem_p)
        cp_q = pltpu.async_copy(q_hbm.at[ridx], qrow, sem_q)
        cp_p.wait()
        cp_q.wait()
        pltpu.sync_copy(prow, pg_hbm.at[pl.ds(off, _C)])
        pltpu.sync_copy(qrow, qg_hbm.at[pl.ds(off, _C)])
        return carry

    lax.fori_loop(0, _NCHUNK, body, 0)


def _edge_body(pg_ref, qg_ref, rel_ref, w1c_ref, w2_ref, b2_ref, out_ref):
    x = (pg_ref[...] + qg_ref[...]
         + jnp.dot(rel_ref[...], w1c_ref[...], preferred_element_type=jnp.float32))
    h = jnp.maximum(x, 0.0)
    out_ref[...] = (jnp.dot(h, w2_ref[...], preferred_element_type=jnp.float32)
                    + b2_ref[...])


def _scatter_body(eff_hbm, rcv_hbm, zeros_hbm, out_hbm, ridx, erow, acc, sem):
    c = lax.axis_index("c")
    s = lax.axis_index("s")
    wid = s * _NC + c

    pltpu.sync_copy(zeros_hbm.at[pl.ds(s * _ROWS_PER_TILE, _ROWS_PER_TILE)],
                    acc.at[pl.ds(s * _ROWS_PER_TILE, _ROWS_PER_TILE)])

    @pl.when(s == _NS - 1)
    def _zero_tail():
        pltpu.sync_copy(zeros_hbm.at[pl.ds(_NS * _ROWS_PER_TILE, _TAIL_ROWS)],
                        acc.at[pl.ds(_NS * _ROWS_PER_TILE, _TAIL_ROWS)])

    plsc.subcore_barrier()

    base = wid * _EW

    def body(j, carry):
        off = base + j * _C
        pltpu.sync_copy(rcv_hbm.at[pl.ds(off, _C)], ridx)
        pltpu.sync_copy(eff_hbm.at[pl.ds(off, _C)], erow)
        pltpu.sync_copy(erow, acc.at[ridx], add=True)
        return carry

    lax.fori_loop(0, _NCHUNK, body, 0)
    plsc.subcore_barrier()

    pltpu.sync_copy(acc.at[pl.ds(s * _ROWS_PER_TILE, _ROWS_PER_TILE)],
                    out_hbm.at[c, pl.ds(s * _ROWS_PER_TILE, _ROWS_PER_TILE)])

    @pl.when(s == _NS - 1)
    def _flush_tail():
        pltpu.sync_copy(acc.at[pl.ds(_NS * _ROWS_PER_TILE, _TAIL_ROWS)],
                        out_hbm.at[c, pl.ds(_NS * _ROWS_PER_TILE, _TAIL_ROWS)])


def _node_body(u_ref, a0_ref, a1_ref, ew1b_ref, ew2_ref, eb2_ref, out_ref):
    agg = a0_ref[...] + a1_ref[...]
    x = u_ref[...] + jnp.dot(agg, ew1b_ref[...], preferred_element_type=jnp.float32)
    h = jnp.maximum(x, 0.0)
    out_ref[...] = (jnp.dot(h, ew2_ref[...], preferred_element_type=jnp.float32)
                    + eb2_ref[...])


_NODE_BLK = 2000
_EDGE_BLK = 2000


def _full_spec(shape):
    return pl.BlockSpec(shape, lambda i: tuple(0 for _ in shape))


def kernel(objects, relations, senders, receivers,
           rW1, rb1, rW2, rb2,
           oW1, ob1, oW2, ob2,
           eW1, eb1, eW2, eb2):
    f32 = jnp.float32
    rW1a = rW1[:_OD]
    rW1b = rW1[_OD:2 * _OD]
    rW1c = rW1[2 * _OD:]
    eW1a = eW1[:_OD]
    eW1b = eW1[_OD:]
    rb1_2d = rb1.reshape(1, _OD)
    rb2_2d = rb2.reshape(1, _OD)
    eb1_2d = eb1.reshape(1, _OD)
    eb2_2d = eb2.reshape(1, _OD)

    n_grid = _N // _NODE_BLK
    row_spec = pl.BlockSpec((_NODE_BLK, _OD), lambda i: (i, 0))
    P, Q, U = pl.pallas_call(
        _pre_body,
        grid=(n_grid,),
        in_specs=[row_spec, _full_spec((_OD, _OD)), _full_spec((_OD, _OD)),
                  _full_spec((1, _OD)), _full_spec((_OD, _OD)),
                  _full_spec((1, _OD))],
        out_specs=[row_spec, row_spec, row_spec],
        out_shape=[jax.ShapeDtypeStruct((_N, _OD), f32)] * 3,
    )(objects, rW1a, rW1b, rb1_2d, eW1a, eb1_2d)

    mesh = plsc.VectorSubcoreMesh(core_axis_name="c", subcore_axis_name="s")
    gather = functools.partial(
        pl.kernel,
        mesh=mesh,
        out_type=[jax.ShapeDtypeStruct((_E, _OD), f32),
                  jax.ShapeDtypeStruct((_E, _OD), f32)],
        scratch_types=[
            pltpu.VMEM((_C,), jnp.int32),
            pltpu.VMEM((_C,), jnp.int32),
            pltpu.VMEM((_C, _OD), f32),
            pltpu.VMEM((_C, _OD), f32),
            pltpu.SemaphoreType.DMA,
            pltpu.SemaphoreType.DMA,
        ],
    )(_gather_body)
    Pg, Qg = gather(P, Q, senders, receivers)

    e_grid = _E // _EDGE_BLK
    erow_spec = pl.BlockSpec((_EDGE_BLK, _OD), lambda i: (i, 0))
    rel_spec = pl.BlockSpec((_EDGE_BLK, _RD), lambda i: (i, 0))
    eff = pl.pallas_call(
        _edge_body,
        grid=(e_grid,),
        in_specs=[erow_spec, erow_spec, rel_spec, _full_spec((_RD, _OD)),
                  _full_spec((_OD, _OD)), _full_spec((1, _OD))],
        out_specs=erow_spec,
        out_shape=jax.ShapeDtypeStruct((_E, _OD), f32),
    )(Pg, Qg, relations, rW1c, rW2, rb2_2d)

    zeros = jnp.zeros((_N, _OD), f32)
    scatter = functools.partial(
        pl.kernel,
        mesh=mesh,
        out_type=jax.ShapeDtypeStruct((_NC, _N, _OD), f32),
        scratch_types=[
            pltpu.VMEM((_C,), jnp.int32),
            pltpu.VMEM((_C, _OD), f32),
            pltpu.VMEM_SHARED((_N, _OD), f32),
            pltpu.SemaphoreType.DMA,
        ],
    )(_scatter_body)
    agg2 = scatter(eff, receivers, zeros)

    out = pl.pallas_call(
        _node_body,
        grid=(n_grid,),
        in_specs=[row_spec, row_spec, row_spec, _full_spec((_OD, _OD)),
                  _full_spec((_OD, _OD)), _full_spec((1, _OD))],
        out_specs=row_spec,
        out_shape=jax.ShapeDtypeStruct((_N, _OD), f32),
    )(U, agg2[0], agg2[1], eW1b, eW2, eb2_2d)
    return out
